# baseline (speedup 1.0000x reference)
import jax
import jax.numpy as jnp
from jax import lax
from jax.experimental import pallas as pl

_S = 5.0
_LOG2E = 1.4426950408889634
_N = 4096
_B = 128
_NB = _N // _B
_SIGNBIT = 0x80000000


def _body(zr_ref, yr_ref, loss_ref):
    alpha = jnp.float32(_S * _LOG2E)
    sz = zr_ref[...] * alpha
    yy = yr_ref[...]
    acc_s = jnp.zeros((_B, _B), jnp.float32)
    acc_c = jnp.zeros((_B, _B), jnp.float32)
    for bi in range(_NB):
        szi = lax.transpose(sz[:, bi * _B:(bi + 1) * _B], (1, 0))
        yi = lax.transpose(yy[:, bi * _B:(bi + 1) * _B], (1, 0))
        for bj in range(bi, _NB):
            szj = sz[:, bj * _B:(bj + 1) * _B]
            yj = yy[:, bj * _B:(bj + 1) * _B]
            dy = yi - yj
            dz = szj - szi
            sbit = lax.bitcast_convert_type(dy, jnp.uint32) & jnp.uint32(_SIGNBIT)
            a = lax.bitcast_convert_type(
                lax.bitcast_convert_type(dz, jnp.uint32) ^ sbit, jnp.float32)
            vals = jnp.log(1.0 + jnp.exp2(a))
            mask = dy != 0.0
            vals_m = jnp.where(mask, vals, 0.0)
            mask_f = jnp.where(mask, 1.0, 0.0)
            if bi == bj:
                vals_m = vals_m * 0.5
                mask_f = mask_f * 0.5
            acc_s = acc_s + vals_m
            acc_c = acc_c + mask_f
    s = jnp.sum(acc_s, keepdims=True).reshape(1, 1)
    c = jnp.sum(acc_c, keepdims=True).reshape(1, 1)
    loss_ref[...] = jnp.where(c > 0, s / jnp.maximum(c, 1.0), 0.0)


@jax.jit
def kernel(z, y):
    z = z.reshape(-1)
    y = y.reshape(-1)
    loss = pl.pallas_call(
        _body,
        out_shape=jax.ShapeDtypeStruct((1, 1), jnp.float32),
    )(
        z.reshape(1, _N),
        y.reshape(1, _N),
    )
    return loss[0, 0]


# speedup vs baseline: 1.4471x; 1.4471x over previous
import jax
import jax.numpy as jnp
from jax import lax
from jax.experimental import pallas as pl

_S = 5.0
_LOG2E = 1.4426950408889634
_N = 4096
_B = 128
_NB = _N // _B
_SIGNBIT = 0x80000000


def _body(zr_ref, yr_ref, loss_ref):
    alpha = jnp.float32(_S * _LOG2E)
    sz = zr_ref[...] * alpha
    yy = yr_ref[...]
    ones = jnp.ones((1, _B), jnp.float32)
    acc_s = jnp.zeros((1, _B), jnp.float32)
    acc_c = jnp.zeros((1, _B), jnp.float32)
    for bi in range(_NB):
        szi = lax.transpose(sz[:, bi * _B:(bi + 1) * _B], (1, 0))
        yi = lax.transpose(yy[:, bi * _B:(bi + 1) * _B], (1, 0))
        for bj in range(bi, _NB):
            szj = sz[:, bj * _B:(bj + 1) * _B]
            yj = yy[:, bj * _B:(bj + 1) * _B]
            dy = yi - yj
            dz = szj - szi
            sbit = lax.bitcast_convert_type(dy, jnp.uint32) & jnp.uint32(_SIGNBIT)
            a = lax.bitcast_convert_type(
                lax.bitcast_convert_type(dz, jnp.uint32) ^ sbit, jnp.float32)
            vals = jnp.log(1.0 + jnp.exp2(a))
            mask = dy != 0.0
            vals_m = jnp.where(mask, vals, 0.0)
            mask_f = jnp.where(mask, 1.0, 0.0)
            rs = jnp.dot(ones, vals_m, preferred_element_type=jnp.float32)
            rc = jnp.dot(ones, mask_f, preferred_element_type=jnp.float32)
            if bi == bj:
                rs = rs * 0.5
                rc = rc * 0.5
            acc_s = acc_s + rs
            acc_c = acc_c + rc
    s = jnp.sum(acc_s, keepdims=True)
    c = jnp.sum(acc_c, keepdims=True)
    loss_ref[...] = jnp.where(c > 0, s / jnp.maximum(c, 1.0), 0.0)


@jax.jit
def kernel(z, y):
    z = z.reshape(-1)
    y = y.reshape(-1)
    loss = pl.pallas_call(
        _body,
        out_shape=jax.ShapeDtypeStruct((1, 1), jnp.float32),
    )(
        z.reshape(1, _N),
        y.reshape(1, _N),
    )
    return loss[0, 0]
